# Initial kernel scaffold; baseline (speedup 1.0000x reference)
import functools

import jax
import jax.numpy as jnp
from jax import lax
from jax.experimental import pallas as pl
from jax.experimental.pallas import tpu as pltpu
from jax.experimental.pallas import tpu_sc as plsc

N = 50000
NT = 51200
E = 800000
NC = 2
NS = 16
NW = NC * NS
B = 128
K = 196
EP = NW * K * B
RPT = NT // NS
DEGW = 16
IN_DIM = 32
OUT_DIM = 128

_MESH = plsc.VectorSubcoreMesh(core_axis_name="c", subcore_axis_name="s")


def _zero_vmem(ref, nrow, ncol):
    z16 = jnp.zeros((16,), jnp.float32)

    def body(i, carry):
        for j in range(ncol // 16):
            ref[i, pl.ds(j * 16, 16)] = z16
        return carry

    lax.fori_loop(0, nrow, body, 0)


def _zero_table(tab, zer, s):
    def body(r, carry):
        pltpu.sync_copy(zer, tab.at[pl.ds(s * RPT + r * B, B)])
        return carry

    lax.fori_loop(0, RPT // B, body, 0)


def _deg_body(dst_hbm, out_hbm, tab, idx_v, val_v, zer_v):
    c = lax.axis_index("c")
    s = lax.axis_index("s")
    wid = c * NS + s
    one16 = jnp.ones((16,), jnp.float32)

    def fill(i, carry):
        val_v[i, :] = one16
        zer_v[i, :] = jnp.zeros((16,), jnp.float32)
        return carry

    lax.fori_loop(0, B, fill, 0)
    _zero_table(tab, zer_v, s)
    plsc.subcore_barrier()

    pltpu.sync_copy(dst_hbm.at[wid], idx_v)

    def body(g, carry):
        pltpu.sync_copy(val_v, tab.at[idx_v.at[g]], add=True)
        return carry

    lax.fori_loop(0, K, body, 0)
    plsc.subcore_barrier()
    pltpu.sync_copy(tab.at[pl.ds(s * RPT, RPT)], out_hbm.at[c, pl.ds(s * RPT, RPT)])


def _make_deg(interpret=False):
    return pl.kernel(
        _deg_body,
        out_type=jax.ShapeDtypeStruct((NC, NT, DEGW), jnp.float32),
        mesh=_MESH,
        scratch_types=[
            pltpu.VMEM_SHARED((NT, DEGW), jnp.float32),
            pltpu.VMEM((K, B), jnp.int32),
            pltpu.VMEM((B, DEGW), jnp.float32),
            pltpu.VMEM((B, DEGW), jnp.float32),
        ],
        interpret=interpret,
    )


def _scat_body(src_hbm, dst_hbm, xs_hbm, out_hbm, tab, sidx, didx, rows, zer, sem):
    c = lax.axis_index("c")
    s = lax.axis_index("s")
    wid = c * NS + s

    _zero_vmem(zer, B, IN_DIM)
    _zero_table(tab, zer, s)
    plsc.subcore_barrier()

    pltpu.sync_copy(src_hbm.at[wid], sidx)
    pltpu.sync_copy(dst_hbm.at[wid], didx)

    def body(g, carry):
        pltpu.async_copy(xs_hbm.at[sidx.at[g]], rows, sem).wait()
        pltpu.sync_copy(rows, tab.at[didx.at[g]], add=True)
        return carry

    lax.fori_loop(0, K, body, 0)
    plsc.subcore_barrier()
    pltpu.sync_copy(tab.at[pl.ds(s * RPT, RPT)], out_hbm.at[c, pl.ds(s * RPT, RPT)])


def _make_scat(interpret=False):
    return pl.kernel(
        _scat_body,
        out_type=jax.ShapeDtypeStruct((NC, NT, IN_DIM), jnp.float32),
        mesh=_MESH,
        scratch_types=[
            pltpu.VMEM_SHARED((NT, IN_DIM), jnp.float32),
            pltpu.VMEM((K, B), jnp.int32),
            pltpu.VMEM((K, B), jnp.int32),
            pltpu.VMEM((B, IN_DIM), jnp.float32),
            pltpu.VMEM((B, IN_DIM), jnp.float32),
            pltpu.SemaphoreType.DMA,
        ],
        interpret=interpret,
    )


def _dec_body(src_hbm, dst_hbm, z_hbm, out_hbm, sidx, didx, zs, zd, res, sem1, sem2):
    c = lax.axis_index("c")
    s = lax.axis_index("s")
    wid = c * NS + s
    lane = jnp.arange(16, dtype=jnp.int32)

    pltpu.sync_copy(src_hbm.at[wid], sidx)
    pltpu.sync_copy(dst_hbm.at[wid], didx)

    def body(g, carry):
        ca = pltpu.async_copy(z_hbm.at[sidx.at[g]], zs, sem1)
        cb = pltpu.async_copy(z_hbm.at[didx.at[g]], zd, sem2)
        ca.wait()
        cb.wait()

        def grp(t, carry2):
            r = jnp.zeros((16,), jnp.float32)
            for k in range(16):
                e = t * 16 + k
                acc = zs[e, pl.ds(0, 16)] * zd[e, pl.ds(0, 16)]
                for j in range(1, OUT_DIM // 16):
                    acc = acc + zs[e, pl.ds(j * 16, 16)] * zd[e, pl.ds(j * 16, 16)]
                r = jnp.where(lane == k, jnp.sum(acc), r)
            res[pl.ds(t * 16, 16)] = 1.0 / (1.0 + jnp.exp(-r)) + 1e-15
            return carry2

        lax.fori_loop(0, B // 16, grp, 0)
        pltpu.sync_copy(res, out_hbm.at[wid, g])
        return carry

    lax.fori_loop(0, K, body, 0)


def _make_dec(interpret=False):
    return pl.kernel(
        _dec_body,
        out_type=jax.ShapeDtypeStruct((NW, K, B), jnp.float32),
        mesh=_MESH,
        scratch_types=[
            pltpu.VMEM((K, B), jnp.int32),
            pltpu.VMEM((K, B), jnp.int32),
            pltpu.VMEM((B, OUT_DIM), jnp.float32),
            pltpu.VMEM((B, OUT_DIM), jnp.float32),
            pltpu.VMEM((B,), jnp.float32),
            pltpu.SemaphoreType.DMA,
            pltpu.SemaphoreType.DMA,
        ],
        interpret=interpret,
    )


def _dense_body(t_ref, xs_ref, d_ref, w_ref, b_ref, z_ref):
    t = t_ref[0] + t_ref[1] + xs_ref[...]
    y = jnp.dot(t, w_ref[...], preferred_element_type=jnp.float32)
    z_ref[...] = jnp.maximum(y * d_ref[...] + b_ref[...], 0.0)


def _make_dense(interpret=False):
    blk = 1600
    return pl.pallas_call(
        _dense_body,
        grid=(NT // blk,),
        in_specs=[
            pl.BlockSpec((NC, blk, IN_DIM), lambda i: (0, i, 0)),
            pl.BlockSpec((blk, IN_DIM), lambda i: (i, 0)),
            pl.BlockSpec((blk, 1), lambda i: (i, 0)),
            pl.BlockSpec((IN_DIM, OUT_DIM), lambda i: (0, 0)),
            pl.BlockSpec((1, OUT_DIM), lambda i: (0, 0)),
        ],
        out_specs=pl.BlockSpec((blk, OUT_DIM), lambda i: (i, 0)),
        out_shape=jax.ShapeDtypeStruct((NT, OUT_DIM), jnp.float32),
        interpret=interpret,
    )


def _build(interpret=False):
    return (_make_deg(interpret), _make_scat(interpret), _make_dec(interpret),
            _make_dense(interpret))


def kernel(x, edge_index, W, b):
    deg_call, scat_call, dec_call, dense_call = _build()

    src = edge_index[0].astype(jnp.int32)
    dst = edge_index[1].astype(jnp.int32)
    pad = EP - E
    srcp = jnp.concatenate([src, jnp.zeros((pad,), jnp.int32)]).reshape(NW, K, B)
    dstp_s = jnp.concatenate([dst, jnp.full((pad,), N, jnp.int32)]).reshape(NW, K, B)
    dstp_d = jnp.concatenate([dst, jnp.zeros((pad,), jnp.int32)]).reshape(NW, K, B)

    degt = deg_call(dstp_s)
    deg = degt[0, :, 0] + degt[1, :, 0] + 1.0
    dinv = lax.rsqrt(deg)
    x_pad = jnp.concatenate([x, jnp.zeros((NT - N, IN_DIM), jnp.float32)])
    xs = x_pad * dinv[:, None]

    t_tab = scat_call(srcp, dstp_s, xs)
    z = dense_call(t_tab, xs, dinv.reshape(NT, 1), W, b.reshape(1, OUT_DIM))

    outr = dec_call(srcp, dstp_d, z)
    adj_pred = outr.reshape(EP)[:E]
    return (adj_pred, edge_index)


# speedup vs baseline: 11.5355x; 11.5355x over previous
import functools

import jax
import jax.numpy as jnp
from jax import lax
from jax.experimental import pallas as pl
from jax.experimental.pallas import tpu as pltpu
from jax.experimental.pallas import tpu_sc as plsc

N = 50000
NT = 51200
E = 800000
NC = 2
NS = 16
NW = NC * NS
B = 128
K = 196
EP = NW * K * B
CHK = 14
RPT = NT // NS
DEGW = 16
IN_DIM = 32
OUT_DIM = 128

_MESH = plsc.VectorSubcoreMesh(core_axis_name="c", subcore_axis_name="s")
_SC_PARAMS = pltpu.CompilerParams(use_tc_tiling_on_sc=False,
                                  needs_layout_passes=False)


def _zero_vmem(ref, nrow, ncol):
    z16 = jnp.zeros((16,), jnp.float32)

    def body(i, carry):
        for j in range(ncol // 16):
            ref[i, pl.ds(j * 16, 16)] = z16
        return carry

    lax.fori_loop(0, nrow, body, 0)


def _zero_table(tab, zer, s):
    def body(r, carry):
        pltpu.sync_copy(zer, tab.at[pl.ds(s * RPT + r * B, B)])
        return carry

    lax.fori_loop(0, RPT // B, body, 0)


def _deg_body(dst_hbm, out_hbm, tab, idx_v, val_v, zer_v):
    c = lax.axis_index("c")
    s = lax.axis_index("s")
    wid = c * NS + s
    one16 = jnp.ones((16,), jnp.float32)

    def fill(i, carry):
        val_v[i, :] = one16
        zer_v[i, :] = jnp.zeros((16,), jnp.float32)
        return carry

    lax.fori_loop(0, B, fill, 0)
    _zero_table(tab, zer_v, s)
    plsc.subcore_barrier()

    pltpu.sync_copy(dst_hbm.at[wid], idx_v)

    def body(g, carry):
        pltpu.sync_copy(val_v, tab.at[idx_v.at[g]], add=True)
        return carry

    lax.fori_loop(0, K, body, 0)
    plsc.subcore_barrier()
    pltpu.sync_copy(tab.at[pl.ds(s * RPT, RPT)], out_hbm.at[c, pl.ds(s * RPT, RPT)])


def _make_deg(interpret=False):
    return pl.kernel(
        _deg_body,
        out_type=jax.ShapeDtypeStruct((NC, NT, DEGW), jnp.float32),
        mesh=_MESH,
        scratch_types=[
            pltpu.VMEM_SHARED((NT, DEGW), jnp.float32),
            pltpu.VMEM((K, B), jnp.int32),
            pltpu.VMEM((B, DEGW), jnp.float32),
            pltpu.VMEM((B, DEGW), jnp.float32),
        ],
        interpret=interpret,
        compiler_params=_SC_PARAMS,
    )


def _scat_body(src_hbm, dst_hbm, xs_hbm, out_hbm, tab, sidx, didx, rows, zer, sem):
    c = lax.axis_index("c")
    s = lax.axis_index("s")
    wid = c * NS + s

    _zero_vmem(zer, B, IN_DIM)
    _zero_table(tab, zer, s)
    plsc.subcore_barrier()

    def outer(o, carry):
        pltpu.sync_copy(src_hbm.at[wid, pl.ds(o * CHK, CHK)], sidx)
        pltpu.sync_copy(dst_hbm.at[wid, pl.ds(o * CHK, CHK)], didx)

        def body(g, carry2):
            pltpu.async_copy(xs_hbm.at[sidx.at[g]], rows, sem).wait()
            pltpu.sync_copy(rows, tab.at[didx.at[g]], add=True)
            return carry2

        lax.fori_loop(0, CHK, body, 0)
        return carry

    lax.fori_loop(0, K // CHK, outer, 0)
    plsc.subcore_barrier()
    pltpu.sync_copy(tab.at[pl.ds(s * RPT, RPT)], out_hbm.at[c, pl.ds(s * RPT, RPT)])


def _make_scat(interpret=False):
    return pl.kernel(
        _scat_body,
        out_type=jax.ShapeDtypeStruct((NC, NT, IN_DIM), jnp.float32),
        mesh=_MESH,
        scratch_types=[
            pltpu.VMEM_SHARED((NT, IN_DIM), jnp.float32),
            pltpu.VMEM((CHK, B), jnp.int32),
            pltpu.VMEM((CHK, B), jnp.int32),
            pltpu.VMEM((B, IN_DIM), jnp.float32),
            pltpu.VMEM((B, IN_DIM), jnp.float32),
            pltpu.SemaphoreType.DMA,
        ],
        interpret=interpret,
        compiler_params=_SC_PARAMS,
    )


def _dec_body(src_hbm, dst_hbm, z_hbm, out_hbm, sidx, didx, zs, zd, res, sem1, sem2):
    c = lax.axis_index("c")
    s = lax.axis_index("s")
    wid = c * NS + s
    lane = jnp.arange(16, dtype=jnp.int32)

    pltpu.sync_copy(src_hbm.at[wid], sidx)
    pltpu.sync_copy(dst_hbm.at[wid], didx)

    def body(g, carry):
        ca = pltpu.async_copy(z_hbm.at[sidx.at[g]], zs, sem1)
        cb = pltpu.async_copy(z_hbm.at[didx.at[g]], zd, sem2)
        ca.wait()
        cb.wait()

        def grp(t, carry2):
            r = jnp.zeros((16,), jnp.float32)
            for k in range(16):
                e = t * 16 + k
                acc = zs[e, pl.ds(0, 16)] * zd[e, pl.ds(0, 16)]
                for j in range(1, OUT_DIM // 16):
                    acc = acc + zs[e, pl.ds(j * 16, 16)] * zd[e, pl.ds(j * 16, 16)]
                r = jnp.where(lane == k, jnp.sum(acc), r)
            res[pl.ds(t * 16, 16)] = 1.0 / (1.0 + jnp.exp(-r)) + 1e-15
            return carry2

        lax.fori_loop(0, B // 16, grp, 0)
        pltpu.sync_copy(res, out_hbm.at[wid, g])
        return carry

    lax.fori_loop(0, K, body, 0)


def _make_dec(interpret=False):
    return pl.kernel(
        _dec_body,
        out_type=jax.ShapeDtypeStruct((NW, K, B), jnp.float32),
        mesh=_MESH,
        scratch_types=[
            pltpu.VMEM((K, B), jnp.int32),
            pltpu.VMEM((K, B), jnp.int32),
            pltpu.VMEM((B, OUT_DIM), jnp.float32),
            pltpu.VMEM((B, OUT_DIM), jnp.float32),
            pltpu.VMEM((B,), jnp.float32),
            pltpu.SemaphoreType.DMA,
            pltpu.SemaphoreType.DMA,
        ],
        interpret=interpret,
        compiler_params=_SC_PARAMS,
    )


def _dense_body(t_ref, xs_ref, d_ref, w_ref, b_ref, z_ref):
    t = t_ref[0] + t_ref[1] + xs_ref[...]
    y = jnp.dot(t, w_ref[...], preferred_element_type=jnp.float32)
    z_ref[...] = jnp.maximum(y * d_ref[...] + b_ref[...], 0.0)


def _make_dense(interpret=False):
    blk = 1600
    return pl.pallas_call(
        _dense_body,
        grid=(NT // blk,),
        in_specs=[
            pl.BlockSpec((NC, blk, IN_DIM), lambda i: (0, i, 0)),
            pl.BlockSpec((blk, IN_DIM), lambda i: (i, 0)),
            pl.BlockSpec((blk, 1), lambda i: (i, 0)),
            pl.BlockSpec((IN_DIM, OUT_DIM), lambda i: (0, 0)),
            pl.BlockSpec((1, OUT_DIM), lambda i: (0, 0)),
        ],
        out_specs=pl.BlockSpec((blk, OUT_DIM), lambda i: (i, 0)),
        out_shape=jax.ShapeDtypeStruct((NT, OUT_DIM), jnp.float32),
        interpret=interpret,
    )


def _build(interpret=False):
    return (_make_deg(interpret), _make_scat(interpret), _make_dec(interpret),
            _make_dense(interpret))


def kernel(x, edge_index, W, b):
    deg_call, scat_call, dec_call, dense_call = _build()

    src = edge_index[0].astype(jnp.int32)
    dst = edge_index[1].astype(jnp.int32)
    pad = EP - E
    srcp = jnp.concatenate([src, jnp.zeros((pad,), jnp.int32)]).reshape(NW, K, B)
    dstp_s = jnp.concatenate([dst, jnp.full((pad,), N, jnp.int32)]).reshape(NW, K, B)
    dstp_d = jnp.concatenate([dst, jnp.zeros((pad,), jnp.int32)]).reshape(NW, K, B)

    degt = deg_call(dstp_s)
    deg = degt[0, :, 0] + degt[1, :, 0] + 1.0
    dinv = lax.rsqrt(deg)
    x_pad = jnp.concatenate([x, jnp.zeros((NT - N, IN_DIM), jnp.float32)])
    xs = x_pad * dinv[:, None]

    t_tab = scat_call(srcp, dstp_s, xs)
    z = dense_call(t_tab, xs, dinv.reshape(NT, 1), W, b.reshape(1, OUT_DIM))

    outr = dec_call(srcp, dstp_d, z)
    adj_pred = outr.reshape(EP)[:E]
    return (adj_pred, edge_index)
